# Initial kernel scaffold; baseline (speedup 1.0000x reference)
import jax
import jax.numpy as jnp
from jax.experimental import pallas as pl


def kernel(batch_size, family_ids, mu, log_sigma, noise):
    raise NotImplementedError("write your pallas kernel here")


# speedup vs baseline: 1.2521x; 1.2521x over previous
import functools
import math

import jax
import jax.numpy as jnp
from jax import lax
from jax.experimental import pallas as pl
from jax.experimental.pallas import tpu as pltpu
from jax.experimental.pallas import tpu_sc as plsc

_LN10 = math.log(10.0)

_INFO = plsc.get_sparse_core_info()
_NC = _INFO.num_cores
_NS = _INFO.num_subcores
_L = _INFO.num_lanes
_NW = _NC * _NS

_CHUNK = 128


@functools.lru_cache(maxsize=None)
def _build(B: int):
    assert B % (8 * _NW) == 0
    bw = B // _NW
    n_chunks = bw // _CHUNK
    assert n_chunks * _CHUNK == bw

    mesh = plsc.VectorSubcoreMesh(core_axis_name="c", subcore_axis_name="s")

    @functools.partial(
        pl.kernel,
        mesh=mesh,
        out_type=jax.ShapeDtypeStruct((B,), jnp.float32),
        scratch_types=[
            pltpu.VMEM((bw,), jnp.int32),
            pltpu.VMEM((bw,), jnp.float32),
            pltpu.VMEM((bw,), jnp.float32),
            pltpu.VMEM((bw,), jnp.float32),
            pltpu.VMEM((bw,), jnp.float32),
            pltpu.SemaphoreType.DMA,
        ],
    )
    def k(fid_hbm, mu_hbm, ls_hbm, nz_hbm, out_hbm,
          idx_v, mu_v, ls_v, nz_v, out_v, sem):
        wid = lax.axis_index("s") * _NC + lax.axis_index("c")
        base = wid * bw
        pltpu.sync_copy(fid_hbm.at[pl.ds(base, bw)], idx_v)
        pltpu.sync_copy(nz_hbm.at[pl.ds(base, bw)], nz_v)

        copies = []
        for c in range(n_chunks):
            sl = pl.ds(c * _CHUNK, _CHUNK)
            copies.append(
                pltpu.async_copy(mu_hbm.at[idx_v.at[sl]], mu_v.at[sl], sem))
            copies.append(
                pltpu.async_copy(ls_hbm.at[idx_v.at[sl]], ls_v.at[sl], sem))
        for cp in copies:
            cp.wait()

        for j in range(bw // _L):
            sl = pl.ds(j * _L, _L)
            sigma = jnp.exp(ls_v[sl])
            out_v[sl] = jnp.exp((mu_v[sl] + sigma * nz_v[sl]) * _LN10)

        pltpu.sync_copy(out_v, out_hbm.at[pl.ds(base, bw)])

    return k


def kernel(batch_size, family_ids, mu, log_sigma, noise):
    B = family_ids.shape[0]
    fids = family_ids.astype(jnp.int32)
    return _build(B)(
        fids,
        mu.astype(jnp.float32),
        log_sigma.astype(jnp.float32),
        noise.astype(jnp.float32),
    )


# speedup vs baseline: 1.2827x; 1.0244x over previous
import functools
import math

import jax
import jax.numpy as jnp
from jax import lax
from jax.experimental import pallas as pl
from jax.experimental.pallas import tpu as pltpu
from jax.experimental.pallas import tpu_sc as plsc

_LN10 = math.log(10.0)

_INFO = plsc.get_sparse_core_info()
_NC = _INFO.num_cores
_NS = _INFO.num_subcores
_L = _INFO.num_lanes
_NW = _NC * _NS

_CHUNK = 128


@functools.lru_cache(maxsize=None)
def _build(B: int):
    assert B % (8 * _NW) == 0
    bw = B // _NW
    n_chunks = bw // _CHUNK
    assert n_chunks * _CHUNK == bw

    mesh = plsc.VectorSubcoreMesh(core_axis_name="c", subcore_axis_name="s")

    @functools.partial(
        pl.kernel,
        mesh=mesh,
        out_type=jax.ShapeDtypeStruct((B,), jnp.float32),
        scratch_types=[
            pltpu.VMEM((bw,), jnp.int32),
            pltpu.VMEM((bw,), jnp.float32),
            pltpu.VMEM((bw,), jnp.float32),
            pltpu.VMEM((bw,), jnp.float32),
            pltpu.VMEM((bw,), jnp.float32),
            pltpu.SemaphoreType.DMA,
            pltpu.SemaphoreType.DMA,
        ],
    )
    def k(fid_hbm, mu_hbm, ls_hbm, nz_hbm, out_hbm,
          idx_v, mu_v, ls_v, nz_v, out_v, sem, nz_sem):
        wid = lax.axis_index("s") * _NC + lax.axis_index("c")
        base = wid * bw
        pltpu.sync_copy(fid_hbm.at[pl.ds(base, bw)], idx_v)
        nz_cp = pltpu.async_copy(nz_hbm.at[pl.ds(base, bw)], nz_v, nz_sem)

        copies = []
        for c in range(n_chunks):
            sl = pl.ds(c * _CHUNK, _CHUNK)
            copies.append(
                pltpu.async_copy(mu_hbm.at[idx_v.at[sl]], mu_v.at[sl], sem))
            copies.append(
                pltpu.async_copy(ls_hbm.at[idx_v.at[sl]], ls_v.at[sl], sem))
        nz_cp.wait()
        vregs_per_chunk = _CHUNK // _L
        for c in range(n_chunks):
            copies[2 * c].wait()
            copies[2 * c + 1].wait()
            for j in range(vregs_per_chunk):
                sl = pl.ds(c * _CHUNK + j * _L, _L)
                sigma = jnp.exp(ls_v[sl])
                out_v[sl] = jnp.exp((mu_v[sl] + sigma * nz_v[sl]) * _LN10)

        pltpu.sync_copy(out_v, out_hbm.at[pl.ds(base, bw)])

    return k


def kernel(batch_size, family_ids, mu, log_sigma, noise):
    B = family_ids.shape[0]
    fids = family_ids.astype(jnp.int32)
    return _build(B)(
        fids,
        mu.astype(jnp.float32),
        log_sigma.astype(jnp.float32),
        noise.astype(jnp.float32),
    )


# speedup vs baseline: 1.2884x; 1.0045x over previous
import functools
import math

import jax
import jax.numpy as jnp
from jax import lax
from jax.experimental import pallas as pl
from jax.experimental.pallas import tpu as pltpu
from jax.experimental.pallas import tpu_sc as plsc

_LN10 = math.log(10.0)

_INFO = plsc.get_sparse_core_info()
_NC = _INFO.num_cores
_NS = _INFO.num_subcores
_L = _INFO.num_lanes
_NW = _NC * _NS

_CHUNK = 128


@functools.lru_cache(maxsize=None)
def _build(B: int):
    assert B % (8 * _NW) == 0
    bw = B // _NW
    n_chunks = bw // _CHUNK
    assert n_chunks * _CHUNK == bw

    mesh = plsc.VectorSubcoreMesh(core_axis_name="c", subcore_axis_name="s")

    @functools.partial(
        pl.kernel,
        mesh=mesh,
        out_type=jax.ShapeDtypeStruct((B,), jnp.float32),
        scratch_types=[
            pltpu.VMEM((bw,), jnp.int32),
            pltpu.VMEM((bw,), jnp.float32),
            pltpu.VMEM((bw,), jnp.float32),
            pltpu.VMEM((bw,), jnp.float32),
            pltpu.VMEM((bw,), jnp.float32),
            pltpu.SemaphoreType.DMA,
            pltpu.SemaphoreType.DMA,
            pltpu.SemaphoreType.DMA,
            pltpu.SemaphoreType.DMA,
        ],
    )
    def k(fid_hbm, mu_hbm, ls_hbm, nz_hbm, out_hbm,
          idx_v, mu_v, ls_v, nz_v, out_v, idx_sem, g_sem, nz_sem, o_sem):
        wid = lax.axis_index("s") * _NC + lax.axis_index("c")
        base = wid * bw

        idx_cps = []
        for c in range(n_chunks):
            sl = pl.ds(c * _CHUNK, _CHUNK)
            idx_cps.append(pltpu.async_copy(
                fid_hbm.at[pl.ds(base + c * _CHUNK, _CHUNK)],
                idx_v.at[sl], idx_sem))
        nz_cp = pltpu.async_copy(nz_hbm.at[pl.ds(base, bw)], nz_v, nz_sem)

        g_cps = []
        for c in range(n_chunks):
            idx_cps[c].wait()
            sl = pl.ds(c * _CHUNK, _CHUNK)
            g_cps.append(
                pltpu.async_copy(mu_hbm.at[idx_v.at[sl]], mu_v.at[sl], g_sem))
            g_cps.append(
                pltpu.async_copy(ls_hbm.at[idx_v.at[sl]], ls_v.at[sl], g_sem))
        nz_cp.wait()

        vregs_per_chunk = _CHUNK // _L
        out_cps = []
        for c in range(n_chunks):
            g_cps[2 * c].wait()
            g_cps[2 * c + 1].wait()
            for j in range(vregs_per_chunk):
                sl = pl.ds(c * _CHUNK + j * _L, _L)
                sigma = jnp.exp(ls_v[sl])
                out_v[sl] = jnp.exp((mu_v[sl] + sigma * nz_v[sl]) * _LN10)
            out_cps.append(pltpu.async_copy(
                out_v.at[pl.ds(c * _CHUNK, _CHUNK)],
                out_hbm.at[pl.ds(base + c * _CHUNK, _CHUNK)], o_sem))
        for cp in out_cps:
            cp.wait()

    return k


def kernel(batch_size, family_ids, mu, log_sigma, noise):
    B = family_ids.shape[0]
    fids = family_ids.astype(jnp.int32)
    return _build(B)(
        fids,
        mu.astype(jnp.float32),
        log_sigma.astype(jnp.float32),
        noise.astype(jnp.float32),
    )
